# baseline (speedup 1.0000x reference)
import functools

import jax
import jax.numpy as jnp
from jax import lax
from jax.experimental import pallas as pl
from jax.experimental.pallas import tpu as pltpu
from jax.experimental.pallas import tpu_sc as plsc

N = 10000
E = 320000
D = 128
H = 128

NC = 2
NS = 16
N_PAD = 10240
CH = 80

EPT_A = E // (NC * NS)
NCH_A = EPT_A // CH
RPT_A = N_PAD // (NC * NS)
EPT_C = E // NS
CH_C = 80
NCH_C = EPT_C // CH_C
LOOP_C = (NCH_C + 3) // 4
ROWS_OUT = N_PAD // NS

_mesh = plsc.VectorSubcoreMesh(core_axis_name="c", subcore_axis_name="s",
                               num_cores=NC, num_subcores=NS)


@functools.partial(
    pl.kernel,
    out_type=(
        jax.ShapeDtypeStruct((NC * N_PAD, D), jnp.float32),
        jax.ShapeDtypeStruct((N_PAD, D), jnp.float32),
    ),
    mesh=_mesh,
    scratch_types=[
        pltpu.VMEM((CH,), jnp.int32),
        pltpu.VMEM((CH,), jnp.int32),
        pltpu.VMEM((CH, D), jnp.float32),
        pltpu.VMEM((CH,), jnp.int32),
        pltpu.VMEM((CH, D), jnp.float32),
        pltpu.VMEM_SHARED((N_PAD, D), jnp.float32),
        pltpu.SemaphoreType.DMA,
        pltpu.SemaphoreType.DMA,
        pltpu.SemaphoreType.DMA,
        pltpu.SemaphoreType.DMA,
    ],
)
def _sc_hist_perm(dst_hbm, perm_hbm, x_hbm, ones_hbm, zerosD_hbm,
                  hist_out, xp_out,
                  didx0, didx1, ones_v, pidx, rows_v, hacc,
                  sem_i0, sem_i1, sem_p, sem_g):
    c = lax.axis_index("c")
    s = lax.axis_index("s")
    w = c * NS + s

    pltpu.sync_copy(zerosD_hbm, hacc.at[pl.ds(s * ROWS_OUT, ROWS_OUT)])
    pltpu.sync_copy(ones_hbm, ones_v)
    plsc.subcore_barrier()

    base_e = w * EPT_A

    def issue_idx(k, buf, sem):
        pltpu.async_copy(dst_hbm.at[pl.ds(base_e + k * CH, CH)], buf, sem)

    def wait_idx(buf, sem):
        pltpu.make_async_copy(dst_hbm.at[pl.ds(0, CH)], buf, sem).wait()

    issue_idx(0, didx0, sem_i0)
    issue_idx(1, didx1, sem_i1)

    def body(j, carry):
        k0 = 2 * j
        wait_idx(didx0, sem_i0)
        pltpu.sync_copy(ones_v, hacc.at[didx0], add=True)

        @pl.when(k0 + 2 < NCH_A)
        def _():
            issue_idx(k0 + 2, didx0, sem_i0)

        wait_idx(didx1, sem_i1)
        pltpu.sync_copy(ones_v, hacc.at[didx1], add=True)

        @pl.when(k0 + 3 < NCH_A)
        def _():
            issue_idx(k0 + 3, didx1, sem_i1)

        return carry

    lax.fori_loop(0, NCH_A // 2, body, 0)
    wait_idx(didx0, sem_i0)
    pltpu.sync_copy(ones_v, hacc.at[didx0], add=True)

    def pbody(q, carry):
        rbase = w * RPT_A + q * CH
        pltpu.async_copy(perm_hbm.at[pl.ds(rbase, CH)], pidx, sem_p).wait()
        pltpu.async_copy(x_hbm.at[pidx], rows_v, sem_g).wait()
        pltpu.sync_copy(rows_v, xp_out.at[pl.ds(rbase, CH)])
        return carry

    lax.fori_loop(0, RPT_A // CH, pbody, 0)

    plsc.subcore_barrier()
    pltpu.sync_copy(hacc.at[pl.ds(s * ROWS_OUT, ROWS_OUT)],
                    hist_out.at[pl.ds(c * N_PAD + s * ROWS_OUT, ROWS_OUT)])


@functools.partial(
    pl.kernel,
    out_type=jax.ShapeDtypeStruct((NC * N_PAD, D), jnp.float32),
    mesh=_mesh,
    scratch_types=[
        pltpu.VMEM((4, CH_C), jnp.int32),
        pltpu.VMEM((4, CH_C), jnp.int32),
        pltpu.VMEM((CH_C, D), jnp.float32),
        pltpu.VMEM((CH_C, D), jnp.float32),
        pltpu.VMEM((CH_C, D), jnp.float32),
        pltpu.VMEM((CH_C, D), jnp.float32),
        pltpu.VMEM_SHARED((N_PAD, D), jnp.float32),
        pltpu.SemaphoreType.DMA,
        pltpu.SemaphoreType.DMA,
        pltpu.SemaphoreType.DMA,
        pltpu.SemaphoreType.DMA,
        pltpu.SemaphoreType.DMA,
        pltpu.SemaphoreType.DMA,
        pltpu.SemaphoreType.DMA,
        pltpu.SemaphoreType.DMA,
        pltpu.SemaphoreType.DMA,
        pltpu.SemaphoreType.DMA,
        pltpu.SemaphoreType.DMA,
        pltpu.SemaphoreType.DMA,
    ],
)
def _sc_aggregate(src2_hbm, dst_hbm, t_hbm, zeros_hbm, g_out,
                  sidx, didx, rows0, rows1, rows2, rows3, acc,
                  sem_i0, sem_i1, sem_i2, sem_i3,
                  sem_g0, sem_g1, sem_g2, sem_g3,
                  sem_c0, sem_c1, sem_c2, sem_c3):
    c = lax.axis_index("c")
    s = lax.axis_index("s")

    rows = (rows0, rows1, rows2, rows3)
    sem_i = (sem_i0, sem_i1, sem_i2, sem_i3)
    sem_g = (sem_g0, sem_g1, sem_g2, sem_g3)
    sem_c = (sem_c0, sem_c1, sem_c2, sem_c3)

    pltpu.sync_copy(zeros_hbm, acc.at[pl.ds(s * ROWS_OUT, ROWS_OUT)])
    plsc.subcore_barrier()

    base_e = s * EPT_C
    src_off = c * E

    def issue_idx(k, m):
        pltpu.async_copy(src2_hbm.at[pl.ds(src_off + base_e + k * CH_C, CH_C)],
                         sidx.at[m], sem_i[m])
        pltpu.async_copy(dst_hbm.at[pl.ds(base_e + k * CH_C, CH_C)],
                         didx.at[m], sem_i[m])

    def wait_idx(m):
        pltpu.make_async_copy(dst_hbm.at[pl.ds(0, CH_C)], sidx.at[m],
                              sem_i[m]).wait()
        pltpu.make_async_copy(dst_hbm.at[pl.ds(0, CH_C)], didx.at[m],
                              sem_i[m]).wait()

    def issue_gather(m):
        pltpu.async_copy(t_hbm.at[sidx.at[m]], rows[m], sem_g[m])

    def wait_gather(m):
        pltpu.make_async_copy(t_hbm.at[pl.ds(0, CH_C)], rows[m],
                              sem_g[m]).wait()

    def issue_scatter(m):
        pltpu.async_copy(rows[m], acc.at[didx.at[m]], sem_c[m], add=True)

    def wait_scatter(m):
        pltpu.make_async_copy(rows[m], acc.at[didx.at[m]], sem_c[m]).wait()

    issue_idx(0, 0)
    issue_idx(1, 1)
    wait_idx(0)
    issue_gather(0)

    def body(j, carry):
        k0 = 4 * j
        for m in range(4):
            k = k0 + m
            p = (m + 2) % 4

            @pl.when(k < NCH_C)
            def _():
                wait_gather(m)
                issue_scatter(m)

            @pl.when(k >= 2)
            def _():
                wait_scatter(p)

            @pl.when(k + 2 < NCH_C)
            def _():
                issue_idx(k + 2, p)

            @pl.when(k + 1 < NCH_C)
            def _():
                wait_idx((m + 1) % 4)
                issue_gather((m + 1) % 4)

        return carry

    lax.fori_loop(0, LOOP_C, body, 0)

    plsc.subcore_barrier()
    pltpu.sync_copy(acc.at[pl.ds(s * ROWS_OUT, ROWS_OUT)],
                    g_out.at[pl.ds(c * N_PAD + s * ROWS_OUT, ROWS_OUT)])


def _tc_prep_body(hist_ref, x_ref, xp_ref, t_ref, nf_ref):
    deg = hist_ref[0:N, :] + hist_ref[N_PAD:N_PAD + N, :] + 1.0
    nf = lax.rsqrt(deg)
    nf_ref[...] = nf
    t_ref[0:N, :] = x_ref[...] * nf
    t_ref[N:2 * N, :] = xp_ref[0:N, :] * nf


_tc_prep = pl.pallas_call(
    _tc_prep_body,
    out_shape=(
        jax.ShapeDtypeStruct((2 * N, D), jnp.float32),
        jax.ShapeDtypeStruct((N, D), jnp.float32),
    ),
)


def _tc_layer1_body(g_ref, t_ref, nf_ref, W1_ref, b1_ref, t2_ref):
    nf = nf_ref[...]
    for i in (0, 1):
        u = nf * (g_ref[i * N_PAD:i * N_PAD + N, :] + t_ref[i * N:(i + 1) * N, :])
        h = jnp.dot(u, W1_ref[...], preferred_element_type=jnp.float32)
        h = jnp.maximum(h + b1_ref[...], 0.0)
        t2_ref[i * N:(i + 1) * N, :] = h * nf


_tc_layer1 = pl.pallas_call(
    _tc_layer1_body,
    out_shape=jax.ShapeDtypeStruct((2 * N, D), jnp.float32),
)


def _tc_final_body(g_ref, t2_ref, nf_ref, W2_ref, b2_ref, w_ref, out_ref):
    nf = nf_ref[...]
    u = nf * (g_ref[0:N, :] + t2_ref[0:N, :])
    z_pos = jnp.dot(u, W2_ref[...], preferred_element_type=jnp.float32) + b2_ref[...]
    u = nf * (g_ref[N_PAD:N_PAD + N, :] + t2_ref[N:2 * N, :])
    z_neg = jnp.dot(u, W2_ref[...], preferred_element_type=jnp.float32) + b2_ref[...]

    summary = jax.nn.sigmoid(jnp.mean(z_pos, axis=0, keepdims=True))

    zw = jnp.dot(z_pos, w_ref[...], preferred_element_type=jnp.float32)
    v_pos = jnp.sum(zw * summary, axis=1, keepdims=True)
    zw = jnp.dot(z_neg, w_ref[...], preferred_element_type=jnp.float32)
    v_neg = jnp.sum(zw * summary, axis=1, keepdims=True)

    pos_loss = -jnp.mean(jnp.log(jax.nn.sigmoid(v_pos) + 1e-15))
    neg_loss = -jnp.mean(jnp.log(1.0 - jax.nn.sigmoid(v_neg) + 1e-15))
    out_ref[...] = jnp.full((1, 1), pos_loss + neg_loss, jnp.float32)


_tc_final = pl.pallas_call(
    _tc_final_body,
    out_shape=jax.ShapeDtypeStruct((1, 1), jnp.float32),
)


def kernel(x, edge_index, batch, perm, W1, b1, W2, b2, w):
    del batch
    src = edge_index[0].astype(jnp.int32)
    dst = edge_index[1].astype(jnp.int32)
    src2 = jnp.concatenate([src, src + N])
    perm_pad = jnp.concatenate(
        [perm.astype(jnp.int32), jnp.zeros((N_PAD - N,), jnp.int32)])
    onesD = jnp.ones((CH, D), jnp.float32)
    zerosD = jnp.zeros((ROWS_OUT, D), jnp.float32)

    hist, xp = _sc_hist_perm(dst, perm_pad, x, onesD, zerosD)
    t1, nf = _tc_prep(hist, x, xp)
    g1 = _sc_aggregate(src2, dst, t1, zerosD)
    t2 = _tc_layer1(g1, t1, nf, W1, b1.reshape(1, H))
    g2 = _sc_aggregate(src2, dst, t2, zerosD)
    loss = _tc_final(g2, t2, nf, W2, b2.reshape(1, H), w)
    return loss[0, 0]


# speedup vs baseline: 1.1736x; 1.1736x over previous
import functools

import jax
import jax.numpy as jnp
from jax import lax
from jax.experimental import pallas as pl
from jax.experimental.pallas import tpu as pltpu
from jax.experimental.pallas import tpu_sc as plsc

N = 10000
E = 320000
D = 128
H = 128

NC = 2
NS = 16
N_PAD = 10240
CH = 80

EPT_A = E // (NC * NS)
CH_A = 128
NCH_A = EPT_A // CH_A
TAIL_A = EPT_A - NCH_A * CH_A
RPT_A = N_PAD // (NC * NS)
EPT_C = E // NS
CH_C = 128
NCH_C = EPT_C // CH_C
TAIL_C = EPT_C - NCH_C * CH_C
ROWS_OUT = N_PAD // NS

_mesh = plsc.VectorSubcoreMesh(core_axis_name="c", subcore_axis_name="s",
                               num_cores=NC, num_subcores=NS)


@functools.partial(
    pl.kernel,
    out_type=(
        jax.ShapeDtypeStruct((NC * N_PAD, D), jnp.float32),
        jax.ShapeDtypeStruct((N_PAD, D), jnp.float32),
    ),
    mesh=_mesh,
    scratch_types=[
        pltpu.VMEM((CH_A,), jnp.int32),
        pltpu.VMEM((CH_A,), jnp.int32),
        pltpu.VMEM((CH_A, D), jnp.float32),
        pltpu.VMEM((TAIL_A,), jnp.int32),
        pltpu.VMEM((CH,), jnp.int32),
        pltpu.VMEM((CH, D), jnp.float32),
        pltpu.VMEM_SHARED((N_PAD, D), jnp.float32),
        pltpu.SemaphoreType.DMA,
        pltpu.SemaphoreType.DMA,
        pltpu.SemaphoreType.DMA,
        pltpu.SemaphoreType.DMA,
    ],
)
def _sc_hist_perm(dst_hbm, perm_hbm, x_hbm, ones_hbm, zerosD_hbm,
                  hist_out, xp_out,
                  didx0, didx1, ones_v, tidx, pidx, rows_v, hacc,
                  sem_i0, sem_i1, sem_p, sem_g):
    c = lax.axis_index("c")
    s = lax.axis_index("s")
    w = c * NS + s

    pltpu.sync_copy(zerosD_hbm, hacc.at[pl.ds(s * ROWS_OUT, ROWS_OUT)])
    pltpu.sync_copy(ones_hbm, ones_v)
    plsc.subcore_barrier()

    base_e = w * EPT_A

    def issue_idx(k, buf, sem):
        pltpu.async_copy(dst_hbm.at[pl.ds(base_e + k * CH_A, CH_A)], buf, sem)

    def wait_idx(buf, sem):
        pltpu.make_async_copy(dst_hbm.at[pl.ds(0, CH_A)], buf, sem).wait()

    issue_idx(0, didx0, sem_i0)
    issue_idx(1, didx1, sem_i1)

    def body(j, carry):
        k0 = 2 * j
        wait_idx(didx0, sem_i0)
        pltpu.sync_copy(ones_v, hacc.at[didx0], add=True)

        @pl.when(k0 + 2 < NCH_A)
        def _():
            issue_idx(k0 + 2, didx0, sem_i0)

        wait_idx(didx1, sem_i1)
        pltpu.sync_copy(ones_v, hacc.at[didx1], add=True)

        @pl.when(k0 + 3 < NCH_A)
        def _():
            issue_idx(k0 + 3, didx1, sem_i1)

        return carry

    lax.fori_loop(0, NCH_A // 2, body, 0)
    pltpu.async_copy(dst_hbm.at[pl.ds(base_e + NCH_A * CH_A, TAIL_A)],
                     tidx, sem_i0).wait()
    pltpu.sync_copy(ones_v.at[pl.ds(0, TAIL_A)], hacc.at[tidx], add=True)

    def pbody(q, carry):
        rbase = w * RPT_A + q * CH
        pltpu.async_copy(perm_hbm.at[pl.ds(rbase, CH)], pidx, sem_p).wait()
        pltpu.async_copy(x_hbm.at[pidx], rows_v, sem_g).wait()
        pltpu.sync_copy(rows_v, xp_out.at[pl.ds(rbase, CH)])
        return carry

    lax.fori_loop(0, RPT_A // CH, pbody, 0)

    plsc.subcore_barrier()
    pltpu.sync_copy(hacc.at[pl.ds(s * ROWS_OUT, ROWS_OUT)],
                    hist_out.at[pl.ds(c * N_PAD + s * ROWS_OUT, ROWS_OUT)])


@functools.partial(
    pl.kernel,
    out_type=jax.ShapeDtypeStruct((NC * N_PAD, D), jnp.float32),
    mesh=_mesh,
    scratch_types=[
        pltpu.VMEM((CH_C,), jnp.int32),
        pltpu.VMEM((CH_C,), jnp.int32),
        pltpu.VMEM((CH_C,), jnp.int32),
        pltpu.VMEM((CH_C,), jnp.int32),
        pltpu.VMEM((CH_C, D), jnp.float32),
        pltpu.VMEM((CH_C, D), jnp.float32),
        pltpu.VMEM((TAIL_C,), jnp.int32),
        pltpu.VMEM((TAIL_C,), jnp.int32),
        pltpu.VMEM((TAIL_C, D), jnp.float32),
        pltpu.VMEM_SHARED((N_PAD, D), jnp.float32),
        pltpu.SemaphoreType.DMA,
        pltpu.SemaphoreType.DMA,
        pltpu.SemaphoreType.DMA,
        pltpu.SemaphoreType.DMA,
        pltpu.SemaphoreType.DMA,
        pltpu.SemaphoreType.DMA,
    ],
)
def _sc_aggregate(src2_hbm, dst_hbm, t_hbm, zeros_hbm, g_out,
                  sidx0, sidx1, didx0, didx1, rows0, rows1,
                  sidxT, didxT, rowsT, acc,
                  sem_s0, sem_s1, sem_d0, sem_d1, sem_g0, sem_g1):
    c = lax.axis_index("c")
    s = lax.axis_index("s")

    pltpu.sync_copy(zeros_hbm, acc.at[pl.ds(s * ROWS_OUT, ROWS_OUT)])
    plsc.subcore_barrier()

    base_e = s * EPT_C
    src_off = c * E

    def issue_idx(k, sbuf, dbuf, ssem, dsem, n=CH_C):
        pltpu.async_copy(src2_hbm.at[pl.ds(src_off + base_e + k * CH_C, n)],
                         sbuf, ssem)
        pltpu.async_copy(dst_hbm.at[pl.ds(base_e + k * CH_C, n)], dbuf, dsem)

    def wait_idx(sbuf, dbuf, ssem, dsem, n=CH_C):
        pltpu.make_async_copy(dst_hbm.at[pl.ds(0, n)], sbuf, ssem).wait()
        pltpu.make_async_copy(dst_hbm.at[pl.ds(0, n)], dbuf, dsem).wait()

    def issue_gather(sbuf, rbuf, gsem):
        pltpu.async_copy(t_hbm.at[sbuf], rbuf, gsem)

    def wait_gather(rbuf, gsem, n=CH_C):
        pltpu.make_async_copy(t_hbm.at[pl.ds(0, n)], rbuf, gsem).wait()

    issue_idx(0, sidx0, didx0, sem_s0, sem_d0)
    issue_idx(1, sidx1, didx1, sem_s1, sem_d1)
    wait_idx(sidx0, didx0, sem_s0, sem_d0)
    issue_gather(sidx0, rows0, sem_g0)

    def body(j, carry):
        k0 = 2 * j
        wait_gather(rows0, sem_g0)
        wait_idx(sidx1, didx1, sem_s1, sem_d1)
        issue_gather(sidx1, rows1, sem_g1)
        pltpu.sync_copy(rows0, acc.at[didx0], add=True)

        @pl.when(k0 + 2 < NCH_C)
        def _():
            issue_idx(k0 + 2, sidx0, didx0, sem_s0, sem_d0)

        wait_gather(rows1, sem_g1)

        @pl.when(k0 + 2 < NCH_C)
        def _():
            wait_idx(sidx0, didx0, sem_s0, sem_d0)
            issue_gather(sidx0, rows0, sem_g0)

        pltpu.sync_copy(rows1, acc.at[didx1], add=True)

        @pl.when(k0 + 3 < NCH_C)
        def _():
            issue_idx(k0 + 3, sidx1, didx1, sem_s1, sem_d1)

        return carry

    lax.fori_loop(0, NCH_C // 2, body, 0)

    issue_idx(NCH_C, sidxT, didxT, sem_s0, sem_d0, n=TAIL_C)
    wait_idx(sidxT, didxT, sem_s0, sem_d0, n=TAIL_C)
    pltpu.async_copy(t_hbm.at[sidxT], rowsT, sem_g0).wait()
    pltpu.sync_copy(rowsT, acc.at[didxT], add=True)

    plsc.subcore_barrier()
    pltpu.sync_copy(acc.at[pl.ds(s * ROWS_OUT, ROWS_OUT)],
                    g_out.at[pl.ds(c * N_PAD + s * ROWS_OUT, ROWS_OUT)])


def _tc_prep_body(hist_ref, x_ref, xp_ref, t_ref, nf_ref):
    deg = hist_ref[0:N, :] + hist_ref[N_PAD:N_PAD + N, :] + 1.0
    nf = lax.rsqrt(deg)
    nf_ref[...] = nf
    t_ref[0:N, :] = x_ref[...] * nf
    t_ref[N:2 * N, :] = xp_ref[0:N, :] * nf


_tc_prep = pl.pallas_call(
    _tc_prep_body,
    out_shape=(
        jax.ShapeDtypeStruct((2 * N, D), jnp.float32),
        jax.ShapeDtypeStruct((N, D), jnp.float32),
    ),
)


def _tc_layer1_body(g_ref, t_ref, nf_ref, W1_ref, b1_ref, t2_ref):
    nf = nf_ref[...]
    for i in (0, 1):
        u = nf * (g_ref[i * N_PAD:i * N_PAD + N, :] + t_ref[i * N:(i + 1) * N, :])
        h = jnp.dot(u, W1_ref[...], preferred_element_type=jnp.float32)
        h = jnp.maximum(h + b1_ref[...], 0.0)
        t2_ref[i * N:(i + 1) * N, :] = h * nf


_tc_layer1 = pl.pallas_call(
    _tc_layer1_body,
    out_shape=jax.ShapeDtypeStruct((2 * N, D), jnp.float32),
)


def _tc_final_body(g_ref, t2_ref, nf_ref, W2_ref, b2_ref, w_ref, out_ref):
    nf = nf_ref[...]
    u = nf * (g_ref[0:N, :] + t2_ref[0:N, :])
    z_pos = jnp.dot(u, W2_ref[...], preferred_element_type=jnp.float32) + b2_ref[...]
    u = nf * (g_ref[N_PAD:N_PAD + N, :] + t2_ref[N:2 * N, :])
    z_neg = jnp.dot(u, W2_ref[...], preferred_element_type=jnp.float32) + b2_ref[...]

    summary = jax.nn.sigmoid(jnp.mean(z_pos, axis=0, keepdims=True))

    zw = jnp.dot(z_pos, w_ref[...], preferred_element_type=jnp.float32)
    v_pos = jnp.sum(zw * summary, axis=1, keepdims=True)
    zw = jnp.dot(z_neg, w_ref[...], preferred_element_type=jnp.float32)
    v_neg = jnp.sum(zw * summary, axis=1, keepdims=True)

    pos_loss = -jnp.mean(jnp.log(jax.nn.sigmoid(v_pos) + 1e-15))
    neg_loss = -jnp.mean(jnp.log(1.0 - jax.nn.sigmoid(v_neg) + 1e-15))
    out_ref[...] = jnp.full((1, 1), pos_loss + neg_loss, jnp.float32)


_tc_final = pl.pallas_call(
    _tc_final_body,
    out_shape=jax.ShapeDtypeStruct((1, 1), jnp.float32),
)


def kernel(x, edge_index, batch, perm, W1, b1, W2, b2, w):
    del batch
    src = edge_index[0].astype(jnp.int32)
    dst = edge_index[1].astype(jnp.int32)
    src2 = jnp.concatenate([src, src + N])
    perm_pad = jnp.concatenate(
        [perm.astype(jnp.int32), jnp.zeros((N_PAD - N,), jnp.int32)])
    onesD = jnp.ones((CH_A, D), jnp.float32)
    zerosD = jnp.zeros((ROWS_OUT, D), jnp.float32)

    hist, xp = _sc_hist_perm(dst, perm_pad, x, onesD, zerosD)
    t1, nf = _tc_prep(hist, x, xp)
    g1 = _sc_aggregate(src2, dst, t1, zerosD)
    t2 = _tc_layer1(g1, t1, nf, W1, b1.reshape(1, H))
    g2 = _sc_aggregate(src2, dst, t2, zerosD)
    loss = _tc_final(g2, t2, nf, W2, b2.reshape(1, H), w)
    return loss[0, 0]


# speedup vs baseline: 1.1758x; 1.0019x over previous
import functools

import jax
import jax.numpy as jnp
from jax import lax
from jax.experimental import pallas as pl
from jax.experimental.pallas import tpu as pltpu
from jax.experimental.pallas import tpu_sc as plsc

N = 10000
E = 320000
D = 128
H = 128

NC = 2
NS = 16
N_PAD = 10240
CH = 80

EPT_A = E // (NC * NS)
CH_A = 128
NCH_A = EPT_A // CH_A
TAIL_A = EPT_A - NCH_A * CH_A
RPT_A = N_PAD // (NC * NS)
EPT_C = E // NS
CH_C = 128
NCH_C = EPT_C // CH_C
TAIL_C = EPT_C - NCH_C * CH_C
ROWS_OUT = N_PAD // NS

_mesh = plsc.VectorSubcoreMesh(core_axis_name="c", subcore_axis_name="s",
                               num_cores=NC, num_subcores=NS)


@functools.partial(
    pl.kernel,
    out_type=(
        jax.ShapeDtypeStruct((NC * N_PAD, D), jnp.float32),
        jax.ShapeDtypeStruct((N_PAD, D), jnp.float32),
    ),
    mesh=_mesh,
    scratch_types=[
        pltpu.VMEM((CH_A,), jnp.int32),
        pltpu.VMEM((CH_A,), jnp.int32),
        pltpu.VMEM((CH_A, D), jnp.float32),
        pltpu.VMEM((TAIL_A,), jnp.int32),
        pltpu.VMEM((CH,), jnp.int32),
        pltpu.VMEM((CH, D), jnp.float32),
        pltpu.VMEM_SHARED((N_PAD, D), jnp.float32),
        pltpu.SemaphoreType.DMA,
        pltpu.SemaphoreType.DMA,
        pltpu.SemaphoreType.DMA,
        pltpu.SemaphoreType.DMA,
    ],
)
def _sc_hist_perm(dst_hbm, perm_hbm, x_hbm, ones_hbm, zerosD_hbm,
                  hist_out, xp_out,
                  didx0, didx1, ones_v, tidx, pidx, rows_v, hacc,
                  sem_i0, sem_i1, sem_p, sem_g):
    c = lax.axis_index("c")
    s = lax.axis_index("s")
    w = c * NS + s

    pltpu.sync_copy(zerosD_hbm, hacc.at[pl.ds(s * ROWS_OUT, ROWS_OUT)])
    pltpu.sync_copy(ones_hbm, ones_v)
    plsc.subcore_barrier()

    base_e = w * EPT_A

    def issue_idx(k, buf, sem):
        pltpu.async_copy(dst_hbm.at[pl.ds(base_e + k * CH_A, CH_A)], buf, sem)

    def wait_idx(buf, sem):
        pltpu.make_async_copy(dst_hbm.at[pl.ds(0, CH_A)], buf, sem).wait()

    issue_idx(0, didx0, sem_i0)
    issue_idx(1, didx1, sem_i1)

    def body(j, carry):
        k0 = 2 * j
        wait_idx(didx0, sem_i0)
        pltpu.sync_copy(ones_v, hacc.at[didx0], add=True)

        @pl.when(k0 + 2 < NCH_A)
        def _():
            issue_idx(k0 + 2, didx0, sem_i0)

        wait_idx(didx1, sem_i1)
        pltpu.sync_copy(ones_v, hacc.at[didx1], add=True)

        @pl.when(k0 + 3 < NCH_A)
        def _():
            issue_idx(k0 + 3, didx1, sem_i1)

        return carry

    lax.fori_loop(0, NCH_A // 2, body, 0)
    pltpu.async_copy(dst_hbm.at[pl.ds(base_e + NCH_A * CH_A, TAIL_A)],
                     tidx, sem_i0).wait()
    pltpu.sync_copy(ones_v.at[pl.ds(0, TAIL_A)], hacc.at[tidx], add=True)

    def pbody(q, carry):
        rbase = w * RPT_A + q * CH
        pltpu.async_copy(perm_hbm.at[pl.ds(rbase, CH)], pidx, sem_p).wait()
        pltpu.async_copy(x_hbm.at[pidx], rows_v, sem_g).wait()
        pltpu.sync_copy(rows_v, xp_out.at[pl.ds(rbase, CH)])
        return carry

    lax.fori_loop(0, RPT_A // CH, pbody, 0)

    plsc.subcore_barrier()
    pltpu.sync_copy(hacc.at[pl.ds(s * ROWS_OUT, ROWS_OUT)],
                    hist_out.at[pl.ds(c * N_PAD + s * ROWS_OUT, ROWS_OUT)])


@functools.partial(
    pl.kernel,
    out_type=jax.ShapeDtypeStruct((NC * N_PAD, D), jnp.float32),
    mesh=_mesh,
    scratch_types=[
        pltpu.VMEM((4, CH_C), jnp.int32),
        pltpu.VMEM((4, CH_C), jnp.int32),
        pltpu.VMEM((CH_C, D), jnp.float32),
        pltpu.VMEM((CH_C, D), jnp.float32),
        pltpu.VMEM((TAIL_C,), jnp.int32),
        pltpu.VMEM((TAIL_C,), jnp.int32),
        pltpu.VMEM((TAIL_C, D), jnp.float32),
        pltpu.VMEM_SHARED((N_PAD, D), jnp.float32),
        pltpu.SemaphoreType.DMA,
        pltpu.SemaphoreType.DMA,
        pltpu.SemaphoreType.DMA,
        pltpu.SemaphoreType.DMA,
        pltpu.SemaphoreType.DMA,
        pltpu.SemaphoreType.DMA,
        pltpu.SemaphoreType.DMA,
        pltpu.SemaphoreType.DMA,
    ],
)
def _sc_aggregate(src2_hbm, dst_hbm, t_hbm, zeros_hbm, g_out,
                  sidx, didx, rows0, rows1,
                  sidxT, didxT, rowsT, acc,
                  sem_i0, sem_i1, sem_i2, sem_i3,
                  sem_g0, sem_g1, sem_c0, sem_c1):
    c = lax.axis_index("c")
    s = lax.axis_index("s")

    rows = (rows0, rows1)
    sem_i = (sem_i0, sem_i1, sem_i2, sem_i3)
    sem_g = (sem_g0, sem_g1)
    sem_c = (sem_c0, sem_c1)

    pltpu.sync_copy(zeros_hbm, acc.at[pl.ds(s * ROWS_OUT, ROWS_OUT)])
    plsc.subcore_barrier()

    base_e = s * EPT_C
    src_off = c * E

    def issue_idx(k, m4):
        pltpu.async_copy(src2_hbm.at[pl.ds(src_off + base_e + k * CH_C, CH_C)],
                         sidx.at[m4], sem_i[m4])
        pltpu.async_copy(dst_hbm.at[pl.ds(base_e + k * CH_C, CH_C)],
                         didx.at[m4], sem_i[m4])

    def wait_idx(m4):
        pltpu.make_async_copy(dst_hbm.at[pl.ds(0, CH_C)], sidx.at[m4],
                              sem_i[m4]).wait()
        pltpu.make_async_copy(dst_hbm.at[pl.ds(0, CH_C)], didx.at[m4],
                              sem_i[m4]).wait()

    def issue_gather(m4, m2):
        pltpu.async_copy(t_hbm.at[sidx.at[m4]], rows[m2], sem_g[m2])

    def wait_gather(m2):
        pltpu.make_async_copy(t_hbm.at[pl.ds(0, CH_C)], rows[m2],
                              sem_g[m2]).wait()

    def issue_scatter(m4, m2):
        pltpu.async_copy(rows[m2], acc.at[didx.at[m4]], sem_c[m2], add=True)

    def wait_scatter(m4, m2):
        pltpu.make_async_copy(rows[m2], acc.at[didx.at[m4]], sem_c[m2]).wait()

    issue_idx(0, 0)
    issue_idx(1, 1)
    issue_idx(2, 2)
    wait_idx(0)
    issue_gather(0, 0)

    def body(j, carry):
        k0 = 4 * j
        for m in range(4):
            k = k0 + m
            m2 = m % 2
            wait_gather(m2)
            issue_scatter(m, m2)

            @pl.when(k >= 1)
            def _():
                wait_scatter((m + 3) % 4, (m2 + 1) % 2)

            @pl.when(k + 1 < NCH_C)
            def _():
                wait_idx((m + 1) % 4)
                issue_gather((m + 1) % 4, (m2 + 1) % 2)

            @pl.when(k + 3 < NCH_C)
            def _():
                issue_idx(k + 3, (m + 3) % 4)

        return carry

    lax.fori_loop(0, NCH_C // 4, body, 0)
    wait_scatter((NCH_C - 1) % 4, (NCH_C - 1) % 2)

    tbase = base_e + NCH_C * CH_C
    pltpu.async_copy(src2_hbm.at[pl.ds(src_off + tbase, TAIL_C)], sidxT, sem_i0)
    pltpu.async_copy(dst_hbm.at[pl.ds(tbase, TAIL_C)], didxT, sem_i0)
    pltpu.make_async_copy(dst_hbm.at[pl.ds(0, TAIL_C)], sidxT, sem_i0).wait()
    pltpu.make_async_copy(dst_hbm.at[pl.ds(0, TAIL_C)], didxT, sem_i0).wait()
    pltpu.async_copy(t_hbm.at[sidxT], rowsT, sem_g0).wait()
    pltpu.sync_copy(rowsT, acc.at[didxT], add=True)

    plsc.subcore_barrier()
    pltpu.sync_copy(acc.at[pl.ds(s * ROWS_OUT, ROWS_OUT)],
                    g_out.at[pl.ds(c * N_PAD + s * ROWS_OUT, ROWS_OUT)])


def _tc_prep_body(hist_ref, x_ref, xp_ref, t_ref, nf_ref):
    deg = hist_ref[0:N, :] + hist_ref[N_PAD:N_PAD + N, :] + 1.0
    nf = lax.rsqrt(deg)
    nf_ref[...] = nf
    t_ref[0:N, :] = x_ref[...] * nf
    t_ref[N:2 * N, :] = xp_ref[0:N, :] * nf


_tc_prep = pl.pallas_call(
    _tc_prep_body,
    out_shape=(
        jax.ShapeDtypeStruct((2 * N, D), jnp.float32),
        jax.ShapeDtypeStruct((N, D), jnp.float32),
    ),
)


def _tc_layer1_body(g_ref, t_ref, nf_ref, W1_ref, b1_ref, t2_ref):
    nf = nf_ref[...]
    for i in (0, 1):
        u = nf * (g_ref[i * N_PAD:i * N_PAD + N, :] + t_ref[i * N:(i + 1) * N, :])
        h = jnp.dot(u, W1_ref[...], preferred_element_type=jnp.float32)
        h = jnp.maximum(h + b1_ref[...], 0.0)
        t2_ref[i * N:(i + 1) * N, :] = h * nf


_tc_layer1 = pl.pallas_call(
    _tc_layer1_body,
    out_shape=jax.ShapeDtypeStruct((2 * N, D), jnp.float32),
)


def _tc_final_body(g_ref, t2_ref, nf_ref, W2_ref, b2_ref, w_ref, out_ref):
    nf = nf_ref[...]
    u = nf * (g_ref[0:N, :] + t2_ref[0:N, :])
    z_pos = jnp.dot(u, W2_ref[...], preferred_element_type=jnp.float32) + b2_ref[...]
    u = nf * (g_ref[N_PAD:N_PAD + N, :] + t2_ref[N:2 * N, :])
    z_neg = jnp.dot(u, W2_ref[...], preferred_element_type=jnp.float32) + b2_ref[...]

    summary = jax.nn.sigmoid(jnp.mean(z_pos, axis=0, keepdims=True))

    zw = jnp.dot(z_pos, w_ref[...], preferred_element_type=jnp.float32)
    v_pos = jnp.sum(zw * summary, axis=1, keepdims=True)
    zw = jnp.dot(z_neg, w_ref[...], preferred_element_type=jnp.float32)
    v_neg = jnp.sum(zw * summary, axis=1, keepdims=True)

    pos_loss = -jnp.mean(jnp.log(jax.nn.sigmoid(v_pos) + 1e-15))
    neg_loss = -jnp.mean(jnp.log(1.0 - jax.nn.sigmoid(v_neg) + 1e-15))
    out_ref[...] = jnp.full((1, 1), pos_loss + neg_loss, jnp.float32)


_tc_final = pl.pallas_call(
    _tc_final_body,
    out_shape=jax.ShapeDtypeStruct((1, 1), jnp.float32),
)


def kernel(x, edge_index, batch, perm, W1, b1, W2, b2, w):
    del batch
    src = edge_index[0].astype(jnp.int32)
    dst = edge_index[1].astype(jnp.int32)
    src2 = jnp.concatenate([src, src + N])
    perm_pad = jnp.concatenate(
        [perm.astype(jnp.int32), jnp.zeros((N_PAD - N,), jnp.int32)])
    onesD = jnp.ones((CH_A, D), jnp.float32)
    zerosD = jnp.zeros((ROWS_OUT, D), jnp.float32)

    hist, xp = _sc_hist_perm(dst, perm_pad, x, onesD, zerosD)
    t1, nf = _tc_prep(hist, x, xp)
    g1 = _sc_aggregate(src2, dst, t1, zerosD)
    t2 = _tc_layer1(g1, t1, nf, W1, b1.reshape(1, H))
    g2 = _sc_aggregate(src2, dst, t2, zerosD)
    loss = _tc_final(g2, t2, nf, W2, b2.reshape(1, H), w)
    return loss[0, 0]


# speedup vs baseline: 1.2142x; 1.0326x over previous
import functools

import jax
import jax.numpy as jnp
from jax import lax
from jax.experimental import pallas as pl
from jax.experimental.pallas import tpu as pltpu
from jax.experimental.pallas import tpu_sc as plsc

N = 10000
E = 320000
D = 128
H = 128

NC = 2
NS = 16
N_PAD = 10240
CH = 80

EPT_A = E // (NC * NS)
CH_A = 128
NCH_A = EPT_A // CH_A
TAIL_A = EPT_A - NCH_A * CH_A
RPT_A = N_PAD // (NC * NS)
W_H = 64
EPT_C = E // NS
CH_C = 128
NCH_C = EPT_C // CH_C
TAIL_C = EPT_C - NCH_C * CH_C
ROWS_OUT = N_PAD // NS

_mesh = plsc.VectorSubcoreMesh(core_axis_name="c", subcore_axis_name="s",
                               num_cores=NC, num_subcores=NS)


@functools.partial(
    pl.kernel,
    out_type=(
        jax.ShapeDtypeStruct((NC * N_PAD, W_H), jnp.float32),
        jax.ShapeDtypeStruct((N_PAD, D), jnp.float32),
    ),
    mesh=_mesh,
    scratch_types=[
        pltpu.VMEM((CH_A,), jnp.int32),
        pltpu.VMEM((CH_A,), jnp.int32),
        pltpu.VMEM((CH_A, W_H), jnp.float32),
        pltpu.VMEM((TAIL_A,), jnp.int32),
        pltpu.VMEM((CH,), jnp.int32),
        pltpu.VMEM((CH, D), jnp.float32),
        pltpu.VMEM_SHARED((N_PAD, W_H), jnp.float32),
        pltpu.SemaphoreType.DMA,
        pltpu.SemaphoreType.DMA,
        pltpu.SemaphoreType.DMA,
        pltpu.SemaphoreType.DMA,
    ],
)
def _sc_hist_perm(dst_hbm, perm_hbm, x_hbm, ones_hbm, zerosD_hbm,
                  hist_out, xp_out,
                  didx0, didx1, ones_v, tidx, pidx, rows_v, hacc,
                  sem_i0, sem_i1, sem_p, sem_g):
    c = lax.axis_index("c")
    s = lax.axis_index("s")
    w = c * NS + s

    pltpu.sync_copy(zerosD_hbm, hacc.at[pl.ds(s * ROWS_OUT, ROWS_OUT)])
    pltpu.sync_copy(ones_hbm, ones_v)
    plsc.subcore_barrier()

    base_e = w * EPT_A

    def issue_idx(k, buf, sem):
        pltpu.async_copy(dst_hbm.at[pl.ds(base_e + k * CH_A, CH_A)], buf, sem)

    def wait_idx(buf, sem):
        pltpu.make_async_copy(dst_hbm.at[pl.ds(0, CH_A)], buf, sem).wait()

    issue_idx(0, didx0, sem_i0)
    issue_idx(1, didx1, sem_i1)

    def body(j, carry):
        k0 = 2 * j
        wait_idx(didx0, sem_i0)
        pltpu.sync_copy(ones_v, hacc.at[didx0], add=True)

        @pl.when(k0 + 2 < NCH_A)
        def _():
            issue_idx(k0 + 2, didx0, sem_i0)

        wait_idx(didx1, sem_i1)
        pltpu.sync_copy(ones_v, hacc.at[didx1], add=True)

        @pl.when(k0 + 3 < NCH_A)
        def _():
            issue_idx(k0 + 3, didx1, sem_i1)

        return carry

    lax.fori_loop(0, NCH_A // 2, body, 0)
    pltpu.async_copy(dst_hbm.at[pl.ds(base_e + NCH_A * CH_A, TAIL_A)],
                     tidx, sem_i0).wait()
    pltpu.sync_copy(ones_v.at[pl.ds(0, TAIL_A)], hacc.at[tidx], add=True)

    def pbody(q, carry):
        rbase = w * RPT_A + q * CH
        pltpu.async_copy(perm_hbm.at[pl.ds(rbase, CH)], pidx, sem_p).wait()
        pltpu.async_copy(x_hbm.at[pidx], rows_v, sem_g).wait()
        pltpu.sync_copy(rows_v, xp_out.at[pl.ds(rbase, CH)])
        return carry

    lax.fori_loop(0, RPT_A // CH, pbody, 0)

    plsc.subcore_barrier()
    pltpu.sync_copy(hacc.at[pl.ds(s * ROWS_OUT, ROWS_OUT)],
                    hist_out.at[pl.ds(c * N_PAD + s * ROWS_OUT, ROWS_OUT)])


@functools.partial(
    pl.kernel,
    out_type=jax.ShapeDtypeStruct((NC * N_PAD, D), jnp.float32),
    mesh=_mesh,
    scratch_types=[
        pltpu.VMEM((4, CH_C), jnp.int32),
        pltpu.VMEM((4, CH_C), jnp.int32),
        pltpu.VMEM((CH_C, D), jnp.float32),
        pltpu.VMEM((CH_C, D), jnp.float32),
        pltpu.VMEM((TAIL_C,), jnp.int32),
        pltpu.VMEM((TAIL_C,), jnp.int32),
        pltpu.VMEM((TAIL_C, D), jnp.float32),
        pltpu.VMEM_SHARED((N_PAD, D), jnp.float32),
        pltpu.SemaphoreType.DMA,
        pltpu.SemaphoreType.DMA,
        pltpu.SemaphoreType.DMA,
        pltpu.SemaphoreType.DMA,
        pltpu.SemaphoreType.DMA,
        pltpu.SemaphoreType.DMA,
        pltpu.SemaphoreType.DMA,
        pltpu.SemaphoreType.DMA,
    ],
)
def _sc_aggregate(src2_hbm, dst_hbm, t_hbm, zeros_hbm, g_out,
                  sidx, didx, rows0, rows1,
                  sidxT, didxT, rowsT, acc,
                  sem_i0, sem_i1, sem_i2, sem_i3,
                  sem_g0, sem_g1, sem_c0, sem_c1):
    c = lax.axis_index("c")
    s = lax.axis_index("s")

    rows = (rows0, rows1)
    sem_i = (sem_i0, sem_i1, sem_i2, sem_i3)
    sem_g = (sem_g0, sem_g1)
    sem_c = (sem_c0, sem_c1)

    pltpu.sync_copy(zeros_hbm, acc.at[pl.ds(s * ROWS_OUT, ROWS_OUT)])
    plsc.subcore_barrier()

    base_e = s * EPT_C
    src_off = c * E

    def issue_idx(k, m4):
        pltpu.async_copy(src2_hbm.at[pl.ds(src_off + base_e + k * CH_C, CH_C)],
                         sidx.at[m4], sem_i[m4])
        pltpu.async_copy(dst_hbm.at[pl.ds(base_e + k * CH_C, CH_C)],
                         didx.at[m4], sem_i[m4])

    def wait_idx(m4):
        pltpu.make_async_copy(dst_hbm.at[pl.ds(0, CH_C)], sidx.at[m4],
                              sem_i[m4]).wait()
        pltpu.make_async_copy(dst_hbm.at[pl.ds(0, CH_C)], didx.at[m4],
                              sem_i[m4]).wait()

    def issue_gather(m4, m2):
        pltpu.async_copy(t_hbm.at[sidx.at[m4]], rows[m2], sem_g[m2])

    def wait_gather(m2):
        pltpu.make_async_copy(t_hbm.at[pl.ds(0, CH_C)], rows[m2],
                              sem_g[m2]).wait()

    def issue_scatter(m4, m2):
        pltpu.async_copy(rows[m2], acc.at[didx.at[m4]], sem_c[m2], add=True)

    def wait_scatter(m4, m2):
        pltpu.make_async_copy(rows[m2], acc.at[didx.at[m4]], sem_c[m2]).wait()

    issue_idx(0, 0)
    issue_idx(1, 1)
    issue_idx(2, 2)
    wait_idx(0)
    issue_gather(0, 0)

    def body(j, carry):
        k0 = 4 * j
        for m in range(4):
            k = k0 + m
            m2 = m % 2
            wait_gather(m2)
            issue_scatter(m, m2)

            @pl.when(k >= 1)
            def _():
                wait_scatter((m + 3) % 4, (m2 + 1) % 2)

            @pl.when(k + 1 < NCH_C)
            def _():
                wait_idx((m + 1) % 4)
                issue_gather((m + 1) % 4, (m2 + 1) % 2)

            @pl.when(k + 3 < NCH_C)
            def _():
                issue_idx(k + 3, (m + 3) % 4)

        return carry

    lax.fori_loop(0, NCH_C // 4, body, 0)
    wait_scatter((NCH_C - 1) % 4, (NCH_C - 1) % 2)

    tbase = base_e + NCH_C * CH_C
    pltpu.async_copy(src2_hbm.at[pl.ds(src_off + tbase, TAIL_C)], sidxT, sem_i0)
    pltpu.async_copy(dst_hbm.at[pl.ds(tbase, TAIL_C)], didxT, sem_i0)
    pltpu.make_async_copy(dst_hbm.at[pl.ds(0, TAIL_C)], sidxT, sem_i0).wait()
    pltpu.make_async_copy(dst_hbm.at[pl.ds(0, TAIL_C)], didxT, sem_i0).wait()
    pltpu.async_copy(t_hbm.at[sidxT], rowsT, sem_g0).wait()
    pltpu.sync_copy(rowsT, acc.at[didxT], add=True)

    plsc.subcore_barrier()
    pltpu.sync_copy(acc.at[pl.ds(s * ROWS_OUT, ROWS_OUT)],
                    g_out.at[pl.ds(c * N_PAD + s * ROWS_OUT, ROWS_OUT)])


def _tc_prep_body(hist_ref, x_ref, xp_ref, t_ref, nf_ref):
    deg = hist_ref[0:N, 0:1] + hist_ref[N_PAD:N_PAD + N, 0:1] + 1.0
    nf = lax.rsqrt(deg) * jnp.ones((1, D), jnp.float32)
    nf_ref[...] = nf
    t_ref[0:N, :] = x_ref[...] * nf
    t_ref[N:2 * N, :] = xp_ref[0:N, :] * nf


_tc_prep = pl.pallas_call(
    _tc_prep_body,
    out_shape=(
        jax.ShapeDtypeStruct((2 * N, D), jnp.float32),
        jax.ShapeDtypeStruct((N, D), jnp.float32),
    ),
)


def _tc_layer1_body(g_ref, t_ref, nf_ref, W1_ref, b1_ref, t2_ref):
    nf = nf_ref[...]
    for i in (0, 1):
        u = nf * (g_ref[i * N_PAD:i * N_PAD + N, :] + t_ref[i * N:(i + 1) * N, :])
        h = jnp.dot(u, W1_ref[...], preferred_element_type=jnp.float32)
        h = jnp.maximum(h + b1_ref[...], 0.0)
        t2_ref[i * N:(i + 1) * N, :] = h * nf


_tc_layer1 = pl.pallas_call(
    _tc_layer1_body,
    out_shape=jax.ShapeDtypeStruct((2 * N, D), jnp.float32),
)


def _tc_final_body(g_ref, t2_ref, nf_ref, W2_ref, b2_ref, w_ref, out_ref):
    nf = nf_ref[...]
    u = nf * (g_ref[0:N, :] + t2_ref[0:N, :])
    z_pos = jnp.dot(u, W2_ref[...], preferred_element_type=jnp.float32) + b2_ref[...]
    u = nf * (g_ref[N_PAD:N_PAD + N, :] + t2_ref[N:2 * N, :])
    z_neg = jnp.dot(u, W2_ref[...], preferred_element_type=jnp.float32) + b2_ref[...]

    summary = jax.nn.sigmoid(jnp.mean(z_pos, axis=0, keepdims=True))

    zw = jnp.dot(z_pos, w_ref[...], preferred_element_type=jnp.float32)
    v_pos = jnp.sum(zw * summary, axis=1, keepdims=True)
    zw = jnp.dot(z_neg, w_ref[...], preferred_element_type=jnp.float32)
    v_neg = jnp.sum(zw * summary, axis=1, keepdims=True)

    pos_loss = -jnp.mean(jnp.log(jax.nn.sigmoid(v_pos) + 1e-15))
    neg_loss = -jnp.mean(jnp.log(1.0 - jax.nn.sigmoid(v_neg) + 1e-15))
    out_ref[...] = jnp.full((1, 1), pos_loss + neg_loss, jnp.float32)


_tc_final = pl.pallas_call(
    _tc_final_body,
    out_shape=jax.ShapeDtypeStruct((1, 1), jnp.float32),
)


def kernel(x, edge_index, batch, perm, W1, b1, W2, b2, w):
    del batch
    src = edge_index[0].astype(jnp.int32)
    dst = edge_index[1].astype(jnp.int32)
    src2 = jnp.concatenate([src, src + N])
    perm_pad = jnp.concatenate(
        [perm.astype(jnp.int32), jnp.zeros((N_PAD - N,), jnp.int32)])
    onesH = jnp.ones((CH_A, W_H), jnp.float32)
    zerosH = jnp.zeros((ROWS_OUT, W_H), jnp.float32)
    zerosD = jnp.zeros((ROWS_OUT, D), jnp.float32)

    hist, xp = _sc_hist_perm(dst, perm_pad, x, onesH, zerosH)
    t1, nf = _tc_prep(hist, x, xp)
    g1 = _sc_aggregate(src2, dst, t1, zerosD)
    t2 = _tc_layer1(g1, t1, nf, W1, b1.reshape(1, H))
    g2 = _sc_aggregate(src2, dst, t2, zerosD)
    loss = _tc_final(g2, t2, nf, W2, b2.reshape(1, H), w)
    return loss[0, 0]
